# baseline (speedup 1.0000x reference)
import functools

import jax
import jax.numpy as jnp
from jax import lax
from jax.experimental import pallas as pl
from jax.experimental.pallas import tpu as pltpu
from jax.experimental.pallas import tpu_sc as plsc

_VOCAB = 100000
_MAXLEN = 200
_EMBED_DIM = 64
_BATCH = 4096

_NC = 2
_NS = 16
_NW = _NC * _NS
_BPW = _BATCH // _NW
_H = _MAXLEN // 2


@functools.partial(
    pl.kernel,
    mesh=plsc.VectorSubcoreMesh(core_axis_name="c", subcore_axis_name="s"),
    out_type=jax.ShapeDtypeStruct((_BATCH, _MAXLEN, _EMBED_DIM), jnp.float32),
    scratch_types=[
        pltpu.VMEM((2 * _BPW, _H), jnp.int32),
        pltpu.VMEM((_H, 2 * _EMBED_DIM), jnp.float32),
        pltpu.VMEM((_MAXLEN, _EMBED_DIM), jnp.float32),
        pltpu.SemaphoreType.DMA,
    ],
    compiler_params=pltpu.CompilerParams(use_tc_tiling_on_sc=True),
)
def _gather_kernel(idx_hbm, table_hbm, out_hbm, idx_v, rows_v, rows64_v, gsem):
    wid = lax.axis_index("s") * _NC + lax.axis_index("c")
    pltpu.sync_copy(idx_hbm.at[wid], idx_v)

    def batch_step(j, carry):
        for h in range(2):
            pltpu.async_copy(table_hbm.at[idx_v.at[2 * j + h]], rows_v, gsem).wait()

            def rep_block(rb, c2):
                for rr in range(10):
                    r = rb * 10 + rr
                    for k in range(_EMBED_DIM // 16):
                        rows64_v[h * _H + r, pl.ds(k * 16, 16)] = (
                            rows_v[r, pl.ds(k * 16, 16)]
                        )
                return c2

            lax.fori_loop(0, _H // 10, rep_block, 0)
        pltpu.sync_copy(rows64_v, out_hbm.at[wid * _BPW + j])
        return carry

    lax.fori_loop(0, _BPW, batch_step, 0)


def kernel(x, token_table, pos_table):
    del pos_table
    idx = x.reshape(_NW, 2 * _BPW, _H).astype(jnp.int32)
    table_pad = jnp.pad(token_table, ((0, 0), (0, _EMBED_DIM)))
    return _gather_kernel(idx, table_pad)


# speedup vs baseline: 1.6110x; 1.6110x over previous
import functools

import jax
import jax.numpy as jnp
from jax import lax
from jax.experimental import pallas as pl
from jax.experimental.pallas import tpu as pltpu
from jax.experimental.pallas import tpu_sc as plsc

_VOCAB = 100000
_MAXLEN = 200
_EMBED_DIM = 64
_BATCH = 4096

_NC = 2
_NS = 16
_NW = _NC * _NS
_TOTAL = _BATCH * _MAXLEN
_ROWS_PER_W = _TOTAL // _NW
_CH = 128
_CPW = _ROWS_PER_W // _CH


@functools.partial(
    pl.kernel,
    mesh=plsc.VectorSubcoreMesh(core_axis_name="c", subcore_axis_name="s"),
    out_type=jax.ShapeDtypeStruct((_TOTAL, _EMBED_DIM), jnp.float32),
    scratch_types=[
        pltpu.VMEM((_CPW, _CH), jnp.int32),
        pltpu.VMEM((2, _CH, 2 * _EMBED_DIM), jnp.float32),
        pltpu.VMEM((2, _CH, _EMBED_DIM), jnp.float32),
        pltpu.SemaphoreType.DMA,
        pltpu.SemaphoreType.DMA,
        pltpu.SemaphoreType.DMA,
        pltpu.SemaphoreType.DMA,
    ],
    compiler_params=pltpu.CompilerParams(use_tc_tiling_on_sc=True),
)
def _gather_kernel(idx_hbm, table_hbm, out_hbm, idx_v, rows_v, rows64_v,
                   g0, g1, w0, w1):
    wid = lax.axis_index("s") * _NC + lax.axis_index("c")
    base = wid * _ROWS_PER_W
    gsems = (g0, g1)
    wsems = (w0, w1)

    pltpu.sync_copy(idx_hbm.at[wid], idx_v)

    def fire_gather(j, p):
        pltpu.async_copy(table_hbm.at[idx_v.at[j]], rows_v.at[p], gsems[p])

    def drain_gather(p):
        pltpu.make_async_copy(
            table_hbm.at[pl.ds(0, _CH)], rows_v.at[p], gsems[p]
        ).wait()

    def fire_write(j, p):
        pltpu.async_copy(
            rows64_v.at[p], out_hbm.at[pl.ds(base + j * _CH, _CH)], wsems[p]
        )

    def drain_write(p):
        pltpu.make_async_copy(
            rows64_v.at[p], out_hbm.at[pl.ds(0, _CH)], wsems[p]
        ).wait()

    def repack(p):
        def block(rb, c2):
            for rr in range(8):
                r = rb * 8 + rr
                for k in range(_EMBED_DIM // 16):
                    rows64_v[p, r, pl.ds(k * 16, 16)] = (
                        rows_v[p, r, pl.ds(k * 16, 16)]
                    )
            return c2

        lax.fori_loop(0, _CH // 8, block, 0)

    fire_gather(0, 0)
    fire_gather(1, 1)

    def outer(i, carry):
        for par in range(2):
            j = 2 * i + par
            drain_gather(par)

            @pl.when(j >= 2)
            def _():
                drain_write(par)

            repack(par)

            @pl.when(j + 2 < _CPW)
            def _():
                fire_gather(j + 2, par)

            fire_write(j, par)
        return carry

    lax.fori_loop(0, _CPW // 2, outer, 0)
    drain_write(0)
    drain_write(1)


def kernel(x, token_table, pos_table):
    del pos_table
    idx = x.reshape(_NW, _CPW, _CH).astype(jnp.int32)
    table_pad = jnp.pad(token_table, ((0, 0), (0, _EMBED_DIM)))
    out = _gather_kernel(idx, table_pad)
    return out.reshape(_BATCH, _MAXLEN, _EMBED_DIM)
